# baseline (speedup 1.0000x reference)
import functools

import jax
import jax.numpy as jnp
from jax import lax
from jax.experimental import pallas as pl
from jax.experimental.pallas import tpu as pltpu
from jax.experimental.pallas import tpu_sc as plsc


@functools.lru_cache(maxsize=None)
def _make_gather(N, V, E):
    info = plsc.get_sparse_core_info()
    NC, NS = info.num_cores, info.num_subcores
    NW = NC * NS
    assert N % NW == 0
    per_w = N // NW
    C = 512
    assert per_w % C == 0
    n_chunks = per_w // C
    mesh = plsc.VectorSubcoreMesh(core_axis_name="c", subcore_axis_name="s")

    @functools.partial(
        pl.kernel,
        mesh=mesh,
        out_type=jax.ShapeDtypeStruct((N, E), jnp.float32),
        scratch_types=[
            pltpu.VMEM((C,), jnp.int32),
            pltpu.VMEM((C, E), jnp.float32),
            pltpu.SemaphoreType.DMA,
        ],
        compiler_params=pltpu.CompilerParams(use_tc_tiling_on_sc=False),
    )
    def body(idx_hbm, table_hbm, out_hbm, idx_v, rows_v, sem):
        wid = lax.axis_index("s") * NC + lax.axis_index("c")
        base = wid * per_w

        def step(g, carry):
            off = base + g * C
            pltpu.sync_copy(idx_hbm.at[pl.ds(off, C)], idx_v)
            pltpu.async_copy(table_hbm.at[idx_v], rows_v, sem).wait()
            pltpu.sync_copy(rows_v, out_hbm.at[pl.ds(off, C)])
            return carry

        lax.fori_loop(0, n_chunks, step, 0)

    return body


def kernel(batch_word_indexes, word_embedding):
    B, L = batch_word_indexes.shape
    V, E = word_embedding.shape
    N = B * L
    idx = batch_word_indexes.reshape(N).astype(jnp.int32)
    out = _make_gather(N, V, E)(idx, word_embedding)
    return out.reshape(B, L, E)


# speedup vs baseline: 1.0483x; 1.0483x over previous
import functools

import jax
import jax.numpy as jnp
from jax import lax
from jax.experimental import pallas as pl
from jax.experimental.pallas import tpu as pltpu
from jax.experimental.pallas import tpu_sc as plsc


@functools.lru_cache(maxsize=None)
def _make_gather(N, V, E):
    info = plsc.get_sparse_core_info()
    NC, NS = info.num_cores, info.num_subcores
    NW = NC * NS
    assert N % NW == 0
    per_w = N // NW
    C = 512
    assert per_w % C == 0
    n_chunks = per_w // C
    mesh = plsc.VectorSubcoreMesh(core_axis_name="c", subcore_axis_name="s")

    assert n_chunks % 2 == 0

    @functools.partial(
        pl.kernel,
        mesh=mesh,
        out_type=jax.ShapeDtypeStruct((N, E), jnp.float32),
        scratch_types=[
            pltpu.VMEM((2, C), jnp.int32),
            pltpu.VMEM((2, C, E), jnp.float32),
            pltpu.SemaphoreType.DMA,
            pltpu.SemaphoreType.DMA,
            pltpu.SemaphoreType.DMA,
            pltpu.SemaphoreType.DMA,
        ],
        compiler_params=pltpu.CompilerParams(use_tc_tiling_on_sc=False),
    )
    def body(idx_hbm, table_hbm, out_hbm, idx_v, rows_v, sg0, sg1, ss0, ss1):
        wid = lax.axis_index("s") * NC + lax.axis_index("c")
        base = wid * per_w
        sgs = (sg0, sg1)
        sss = (ss0, ss1)

        def load_idx(g, b):
            pltpu.sync_copy(idx_hbm.at[pl.ds(base + g * C, C)], idx_v.at[b])

        def start_gather(b):
            pltpu.async_copy(table_hbm.at[idx_v.at[b]], rows_v.at[b], sgs[b])

        def wait_gather(b):
            pltpu.make_async_copy(
                table_hbm.at[idx_v.at[b]], rows_v.at[b], sgs[b]
            ).wait()

        def start_store(g, b):
            pltpu.async_copy(
                rows_v.at[b], out_hbm.at[pl.ds(base + g * C, C)], sss[b]
            )

        def wait_store(b):
            pltpu.make_async_copy(
                rows_v.at[b], out_hbm.at[pl.ds(base, C)], sss[b]
            ).wait()

        for b in range(2):
            load_idx(b, b)
            start_gather(b)

        def outer(o, carry):
            for b in range(2):
                g_prev = 2 * (o - 1) + b
                g_cur = 2 * o + b
                wait_gather(b)
                start_store(g_prev, b)
                load_idx(g_cur, b)
                wait_store(b)
                start_gather(b)
            return carry

        lax.fori_loop(1, n_chunks // 2, outer, 0)

        for b in range(2):
            g_prev = n_chunks - 2 + b
            wait_gather(b)
            pltpu.sync_copy(rows_v.at[b], out_hbm.at[pl.ds(base + g_prev * C, C)])

    return body


def kernel(batch_word_indexes, word_embedding):
    B, L = batch_word_indexes.shape
    V, E = word_embedding.shape
    N = B * L
    idx = batch_word_indexes.reshape(N).astype(jnp.int32)
    out = _make_gather(N, V, E)(idx, word_embedding)
    return out.reshape(B, L, E)
